# baseline (speedup 1.0000x reference)
import functools

import jax
import jax.numpy as jnp
from jax import lax
from jax.experimental import pallas as pl
from jax.experimental.pallas import tpu as pltpu
from jax.experimental.pallas import tpu_sc as plsc

E = 6400000
N = 100000
NPAD = 100352
NW = 32
E_PER_W = E // NW
BLK = 2000
NBLK = E_PER_W // BLK
VEC = 16
SUB = BLK // VEC
UNROLL = 5
NBUF = 6


def _sc_partial_sums(x, index):
  mesh = plsc.VectorSubcoreMesh(core_axis_name="c", subcore_axis_name="s")

  @functools.partial(
      pl.kernel,
      mesh=mesh,
      out_type=jax.ShapeDtypeStruct((NW, NPAD), jnp.float32),
      compiler_params=pltpu.CompilerParams(needs_layout_passes=False),
      scratch_types=[
          pltpu.VMEM((NPAD,), jnp.float32),
          [pltpu.VMEM((BLK,), jnp.float32)] * NBUF,
          [pltpu.VMEM((BLK,), jnp.int32)] * NBUF,
          [pltpu.SemaphoreType.DMA] * NBUF,
          [pltpu.SemaphoreType.DMA] * NBUF,
      ],
  )
  def k(x_hbm, idx_hbm, out_hbm, acc, xb, ib, sx, si):
    cid = lax.axis_index("c")
    sid = lax.axis_index("s")
    wid = sid * 2 + cid
    base = wid * E_PER_W

    zeros = jnp.zeros((VEC,), jnp.float32)

    @plsc.parallel_loop(0, NPAD // VEC, unroll=8)
    def _zero(i):
      acc[pl.ds(i * VEC, VEC)] = zeros

    def make_copies(blk, slot):
      off = base + blk * BLK
      cpx = pltpu.make_async_copy(
          x_hbm.at[pl.ds(off, BLK)], xb[slot], sx[slot])
      cpi = pltpu.make_async_copy(
          idx_hbm.at[pl.ds(off, BLK)], ib[slot], si[slot])
      return cpx, cpi

    def start(blk, slot):
      cpx, cpi = make_copies(blk, slot)
      cpx.start()
      cpi.start()

    def wait(blk, slot):
      cpx, cpi = make_copies(blk, slot)
      cpx.wait()
      cpi.wait()

    lanebase = jnp.arange(VEC, dtype=jnp.int32) * SUB

    def process(slot):
      @plsc.parallel_loop(0, SUB, unroll=UNROLL)
      def _scatter(i):
        pos = lanebase + i
        idxv = plsc.load_gather(ib[slot], [pos])
        xv = plsc.load_gather(xb[slot], [pos])
        plsc.addupdate_scatter(acc, [idxv], xv)

    for slot in range(NBUF):
      start(slot, slot)
    for blk in range(NBLK):
      slot = blk % NBUF
      wait(blk, slot)
      process(slot)
      if blk + NBUF < NBLK:
        start(blk + NBUF, slot)

    pltpu.sync_copy(acc, out_hbm.at[wid])

  return k(x, index)


def _tc_reduce(partials):
  CHUNK = 14336

  def body(p_ref, o_ref):
    o_ref[...] = jnp.sum(p_ref[...], axis=0)

  return pl.pallas_call(
      body,
      grid=(NPAD // CHUNK,),
      in_specs=[pl.BlockSpec((NW, CHUNK), lambda i: (0, i))],
      out_specs=pl.BlockSpec((CHUNK,), lambda i: (i,)),
      out_shape=jax.ShapeDtypeStruct((NPAD,), jnp.float32),
  )(partials)


@jax.jit
def kernel(x, index):
  partials = _sc_partial_sums(x, index)
  out = _tc_reduce(partials)
  return out[:N]


# speedup vs baseline: 1.1137x; 1.1137x over previous
import functools

import jax
import jax.numpy as jnp
from jax import lax
from jax.experimental import pallas as pl
from jax.experimental.pallas import tpu as pltpu
from jax.experimental.pallas import tpu_sc as plsc

E = 6400000
N = 100000
NPAD = 100352
NW = 32
E_PER_W = E // NW
BLK = 2000
NBLK = E_PER_W // BLK
VEC = 16
SUB = BLK // VEC
UNROLL = 25
NBUF = 5
NGRP = NBLK // NBUF


def _sc_partial_sums(x, index):
  mesh = plsc.VectorSubcoreMesh(core_axis_name="c", subcore_axis_name="s")

  @functools.partial(
      pl.kernel,
      mesh=mesh,
      out_type=jax.ShapeDtypeStruct((NW, NPAD), jnp.float32),
      compiler_params=pltpu.CompilerParams(needs_layout_passes=False),
      scratch_types=[
          pltpu.VMEM((NPAD,), jnp.float32),
          [pltpu.VMEM((BLK,), jnp.float32)] * NBUF,
          [pltpu.VMEM((BLK,), jnp.int32)] * NBUF,
          [pltpu.SemaphoreType.DMA] * NBUF,
          [pltpu.SemaphoreType.DMA] * NBUF,
      ],
  )
  def k(x_hbm, idx_hbm, out_hbm, acc, xb, ib, sx, si):
    cid = lax.axis_index("c")
    sid = lax.axis_index("s")
    wid = sid * 2 + cid
    base = wid * E_PER_W

    zeros = jnp.zeros((VEC,), jnp.float32)

    @plsc.parallel_loop(0, NPAD // VEC, unroll=8)
    def _zero(i):
      acc[pl.ds(i * VEC, VEC)] = zeros

    def make_copies(blk, slot):
      off = base + blk * BLK
      cpx = pltpu.make_async_copy(
          x_hbm.at[pl.ds(off, BLK)], xb[slot], sx[slot])
      cpi = pltpu.make_async_copy(
          idx_hbm.at[pl.ds(off, BLK)], ib[slot], si[slot])
      return cpx, cpi

    def start(blk, slot):
      cpx, cpi = make_copies(blk, slot)
      cpx.start()
      cpi.start()

    def wait(blk, slot):
      cpx, cpi = make_copies(blk, slot)
      cpx.wait()
      cpi.wait()

    lanebase = jnp.arange(VEC, dtype=jnp.int32) * SUB

    def process(slot):
      @plsc.parallel_loop(0, SUB, unroll=UNROLL)
      def _scatter(i):
        pos = lanebase + i
        idxv = plsc.load_gather(ib[slot], [pos])
        xv = plsc.load_gather(xb[slot], [pos])
        plsc.addupdate_scatter(acc, [idxv], xv)

    for slot in range(NBUF):
      start(slot, slot)

    def group(g, c):
      blk = g * NBUF
      for slot in range(NBUF):
        wait(blk + slot, slot)
        process(slot)
        start(blk + slot + NBUF, slot)
      return c

    lax.fori_loop(0, NGRP - 1, group, 0)
    for slot in range(NBUF):
      wait((NGRP - 1) * NBUF + slot, slot)
      process(slot)

    pltpu.sync_copy(acc, out_hbm.at[wid])

  return k(x, index)


def _tc_reduce(partials):
  CHUNK = 14336

  def body(p_ref, o_ref):
    o_ref[...] = jnp.sum(p_ref[...], axis=0)

  return pl.pallas_call(
      body,
      grid=(NPAD // CHUNK,),
      in_specs=[pl.BlockSpec((NW, CHUNK), lambda i: (0, i))],
      out_specs=pl.BlockSpec((CHUNK,), lambda i: (i,)),
      out_shape=jax.ShapeDtypeStruct((NPAD,), jnp.float32),
  )(partials)


@jax.jit
def kernel(x, index):
  partials = _sc_partial_sums(x, index)
  out = _tc_reduce(partials)
  return out[:N]


# speedup vs baseline: 1.1885x; 1.0672x over previous
import functools

import jax
import jax.numpy as jnp
from jax import lax
from jax.experimental import pallas as pl
from jax.experimental.pallas import tpu as pltpu
from jax.experimental.pallas import tpu_sc as plsc

E = 6400000
N = 100000
NPAD = 100352
NW = 32
E_PER_W = E // NW
BLK = 2000
NBLK = E_PER_W // BLK
VEC = 16
SUB = BLK // VEC
UNROLL = 25
NBUF = 5
NGRP = NBLK // NBUF
WCH = 12544


def _sc_partial_sums(x, index):
  mesh = plsc.VectorSubcoreMesh(core_axis_name="c", subcore_axis_name="s")

  @functools.partial(
      pl.kernel,
      mesh=mesh,
      out_type=(
          jax.ShapeDtypeStruct((NW, NPAD), jnp.float32),
          jax.ShapeDtypeStruct((NW, VEC), jnp.int32),
      ),
      compiler_params=pltpu.CompilerParams(needs_layout_passes=False),
      scratch_types=[
          pltpu.VMEM((NPAD,), jnp.float32),
          [pltpu.VMEM((BLK,), jnp.float32)] * NBUF,
          [pltpu.VMEM((BLK,), jnp.int32)] * NBUF,
          pltpu.VMEM((VEC,), jnp.int32),
          pltpu.VMEM((VEC,), jnp.int32),
          [pltpu.SemaphoreType.DMA] * NBUF,
          [pltpu.SemaphoreType.DMA] * NBUF,
          pltpu.SemaphoreType.DMA,
          pltpu.SemaphoreType.DMA,
      ],
  )
  def k(x_hbm, idx_hbm, out_hbm, bounds_hbm, acc, xb, ib, fb, lb,
        sx, si, sf, sl):
    cid = lax.axis_index("c")
    sid = lax.axis_index("s")
    wid = sid * 2 + cid
    base = wid * E_PER_W

    zeros = jnp.zeros((VEC,), jnp.float32)
    iota16 = lax.iota(jnp.int32, VEC)

    def make_copies(blk, slot):
      off = base + blk * BLK
      cpx = pltpu.make_async_copy(
          x_hbm.at[pl.ds(off, BLK)], xb[slot], sx[slot])
      cpi = pltpu.make_async_copy(
          idx_hbm.at[pl.ds(off, BLK)], ib[slot], si[slot])
      return cpx, cpi

    def start(blk, slot):
      cpx, cpi = make_copies(blk, slot)
      cpx.start()
      cpi.start()

    def wait(blk, slot):
      cpx, cpi = make_copies(blk, slot)
      cpx.wait()
      cpi.wait()

    cpf = pltpu.make_async_copy(idx_hbm.at[pl.ds(base, VEC)], fb, sf)
    cpl = pltpu.make_async_copy(
        idx_hbm.at[pl.ds(base + E_PER_W - VEC, VEC)], lb, sl)
    cpf.start()
    cpl.start()
    for slot in range(NBUF):
      start(slot, slot)
    cpf.wait()
    cpl.wait()
    lo = jnp.min(fb[pl.ds(0, VEC)])
    hi = jnp.max(lb[pl.ds(0, VEC)])

    v0 = lo >> 4
    v1 = hi >> 4

    @plsc.parallel_loop(v0, v1 + 1, unroll=1)
    def _zero(i):
      acc[pl.ds(i * VEC, VEC)] = zeros

    lanebase = jnp.arange(VEC, dtype=jnp.int32) * SUB

    def process(slot):
      @plsc.parallel_loop(0, SUB, unroll=UNROLL)
      def _scatter(i):
        pos = lanebase + i
        idxv = plsc.load_gather(ib[slot], [pos])
        xv = plsc.load_gather(xb[slot], [pos])
        plsc.addupdate_scatter(acc, [idxv], xv)

    def group(g, c):
      blk = g * NBUF
      for slot in range(NBUF):
        wait(blk + slot, slot)
        process(slot)
        start(blk + slot + NBUF, slot)
      return c

    lax.fori_loop(0, NGRP - 1, group, 0)
    for slot in range(NBUF):
      wait((NGRP - 1) * NBUF + slot, slot)
      process(slot)

    fb[pl.ds(0, VEC)] = jnp.where(iota16 == 0, lo, hi)
    pltpu.sync_copy(fb, bounds_hbm.at[wid])
    for c in range(NPAD // WCH):
      @pl.when((lo < (c + 1) * WCH) & (hi >= c * WCH))
      def _writeout():
        pltpu.sync_copy(acc.at[pl.ds(c * WCH, WCH)],
                        out_hbm.at[wid, pl.ds(c * WCH, WCH)])

  return k(x, index)


def _tc_reduce(partials, bounds):
  CHUNK = 14336

  def body(p_ref, b_ref, o_ref):
    i = pl.program_id(0)
    b = b_ref[...]
    lo = b[:, 0:1]
    hi = b[:, 1:2]
    cols = jax.lax.broadcasted_iota(jnp.int32, (NW, CHUNK), 1) + i * CHUNK
    mask = (cols >= lo) & (cols <= hi)
    vals = jnp.where(mask, p_ref[...], 0.0)
    o_ref[...] = jnp.sum(vals, axis=0)

  return pl.pallas_call(
      body,
      grid=(NPAD // CHUNK,),
      in_specs=[
          pl.BlockSpec((NW, CHUNK), lambda i: (0, i)),
          pl.BlockSpec((NW, VEC), lambda i: (0, 0)),
      ],
      out_specs=pl.BlockSpec((CHUNK,), lambda i: (i,)),
      out_shape=jax.ShapeDtypeStruct((NPAD,), jnp.float32),
  )(partials, bounds)


@jax.jit
def kernel(x, index):
  partials, bounds = _sc_partial_sums(x, index)
  out = _tc_reduce(partials, bounds)
  return out[:N]
